# Initial kernel scaffold; baseline (speedup 1.0000x reference)
import jax
import jax.numpy as jnp
from jax.experimental import pallas as pl


def kernel(x, emb_table, W, b):
    raise NotImplementedError("write your pallas kernel here")


# speedup vs baseline: 2.0920x; 2.0920x over previous
import functools

import jax
import jax.numpy as jnp
import numpy as np
from jax import lax
from jax.experimental import pallas as pl
from jax.experimental.pallas import tpu as pltpu
from jax.experimental.pallas import tpu_sc as plsc

B = 4096
SEQ = 200
D = 32
OUT_DIM = 128

NC = 2
NS = 16
NW = NC * NS
RPW = B // NW
IPW = RPW * SEQ
UNIT = 128
UPC = 8
CHUNK_ROWS = UPC * UNIT
CHUNKS = IPW // CHUNK_ROWS
XROWS = (B * SEQ) // UNIT
XR_PW = IPW // UNIT


def _dest_pattern() -> np.ndarray:
    j = np.arange(B * SEQ, dtype=np.int64)
    wid = j // IPW
    jj = j % IPW
    slot = (wid // NC) * RPW + jj // SEQ
    return slot.reshape(XROWS, UNIT).astype(np.int32)


_DEST = _dest_pattern()


def _sc_body(x_hbm, dest_hbm, table_hbm, out_hbm,
             idx_v, dest_v, rows_v, pooled_v, accum_sh, gsem, ssem):
    c = lax.axis_index("c")
    s = lax.axis_index("s")
    wid = s * NC + c

    z = jnp.zeros((16,), jnp.float32)
    for r in range(RPW):
        rows_v[r, 0:16] = z
        rows_v[r, 16:32] = z
    pltpu.sync_copy(rows_v.at[pl.ds(0, RPW)], accum_sh.at[pl.ds(s * RPW, RPW)])

    xbase = wid * XR_PW

    @pl.loop(0, CHUNKS)
    def _chunk(i):
        r0 = xbase + i * UPC
        pltpu.sync_copy(x_hbm.at[pl.ds(r0, UPC)], idx_v)
        pltpu.sync_copy(dest_hbm.at[pl.ds(r0, UPC)], dest_v)
        gathers = [
            pltpu.async_copy(table_hbm.at[idx_v.at[u]],
                             rows_v.at[pl.ds(u * UNIT, UNIT)], gsem)
            for u in range(UPC)
        ]
        for g in gathers:
            g.wait()
        scatters = [
            pltpu.async_copy(rows_v.at[pl.ds(u * UNIT, UNIT)],
                             accum_sh.at[dest_v.at[u]], ssem, add=True)
            for u in range(UPC)
        ]
        for t in scatters:
            t.wait()

    pltpu.sync_copy(accum_sh.at[pl.ds(s * RPW, RPW)], pooled_v)
    pltpu.sync_copy(pooled_v, out_hbm.at[pl.ds(wid * RPW, RPW)])


@functools.partial(jax.jit, static_argnames=())
def _sc_pooled_sums(x2, dest2, table):
    mesh = plsc.VectorSubcoreMesh(core_axis_name="c", subcore_axis_name="s",
                                  num_cores=NC, num_subcores=NS)
    return pl.kernel(
        _sc_body,
        out_type=jax.ShapeDtypeStruct((B, D), jnp.float32),
        mesh=mesh,
        scratch_types=[
            pltpu.VMEM((UPC, UNIT), jnp.int32),
            pltpu.VMEM((UPC, UNIT), jnp.int32),
            pltpu.VMEM((CHUNK_ROWS, D), jnp.float32),
            pltpu.VMEM((RPW, D), jnp.float32),
            pltpu.VMEM_SHARED((NS * RPW, D), jnp.float32),
            pltpu.SemaphoreType.DMA,
            pltpu.SemaphoreType.DMA,
        ],
        compiler_params=pltpu.CompilerParams(use_tc_tiling_on_sc=False),
    )(x2, dest2, table)


def _mm_body(p_ref, w_ref, b_ref, o_ref):
    p = p_ref[...] * (1.0 / SEQ)
    o_ref[...] = lax.dot_general(
        p, w_ref[...], (((1,), (1,)), ((), ())),
        preferred_element_type=jnp.float32) + b_ref[...]


def _classifier(pooled_sums, W, b):
    return pl.pallas_call(
        _mm_body,
        out_shape=jax.ShapeDtypeStruct((B, OUT_DIM), jnp.float32),
    )(pooled_sums, W, b.reshape(1, OUT_DIM))


def kernel(x, emb_table, W, b):
    x2 = x.astype(jnp.int32).reshape(XROWS, UNIT)
    dest2 = jnp.asarray(_DEST)
    pooled_sums = _sc_pooled_sums(x2, dest2, emb_table)
    return _classifier(pooled_sums, W, b)


# speedup vs baseline: 2.1321x; 1.0192x over previous
import jax
import jax.numpy as jnp
from jax import lax
from jax.experimental import pallas as pl
from jax.experimental.pallas import tpu as pltpu
from jax.experimental.pallas import tpu_sc as plsc

B = 4096
SEQ = 200
D = 32
OUT_DIM = 128

NC = 2
NS = 16
NW = NC * NS
RPW = B // NW
IPW = RPW * SEQ
UNIT = 128
UPC = 8
CHUNK_ROWS = UPC * UNIT
CHUNKS = IPW // CHUNK_ROWS


def _sc_body(x_hbm, table_hbm, out_hbm,
             idx_v, dest_v, rows_v, pooled_v, accum_sh, gsem, ssem):
    c = lax.axis_index("c")
    s = lax.axis_index("s")
    wid = s * NC + c

    z = jnp.zeros((16,), jnp.float32)
    for r in range(RPW):
        rows_v[r, 0:16] = z
        rows_v[r, 16:32] = z
    pltpu.sync_copy(rows_v.at[pl.ds(0, RPW)], accum_sh.at[pl.ds(s * RPW, RPW)])

    base0 = wid * IPW
    lane = lax.iota(jnp.int32, 16)
    srow = s * RPW

    @pl.loop(0, CHUNKS)
    def _chunk(i):
        flat0 = base0 + i * CHUNK_ROWS
        pltpu.sync_copy(x_hbm.at[pl.ds(flat0, CHUNK_ROWS)], idx_v)
        for u in range(UPC):
            for k in range(UNIT // 16):
                f = i * CHUNK_ROWS + u * UNIT + k * 16
                dest_v[u, k * 16:(k + 1) * 16] = (
                    srow + lax.div(f + lane, SEQ))
        gathers = [
            pltpu.async_copy(table_hbm.at[idx_v.at[pl.ds(u * UNIT, UNIT)]],
                             rows_v.at[pl.ds(u * UNIT, UNIT)], gsem)
            for u in range(UPC)
        ]
        for g in gathers:
            g.wait()
        scatters = [
            pltpu.async_copy(rows_v.at[pl.ds(u * UNIT, UNIT)],
                             accum_sh.at[dest_v.at[u]], ssem, add=True)
            for u in range(UPC)
        ]
        for t in scatters:
            t.wait()

    pltpu.sync_copy(accum_sh.at[pl.ds(s * RPW, RPW)], pooled_v)
    pltpu.sync_copy(pooled_v, out_hbm.at[pl.ds(wid * RPW, RPW)])


def _sc_pooled_sums(x1, table):
    mesh = plsc.VectorSubcoreMesh(core_axis_name="c", subcore_axis_name="s",
                                  num_cores=NC, num_subcores=NS)
    return pl.kernel(
        _sc_body,
        out_type=jax.ShapeDtypeStruct((B, D), jnp.float32),
        mesh=mesh,
        scratch_types=[
            pltpu.VMEM((CHUNK_ROWS,), jnp.int32),
            pltpu.VMEM((UPC, UNIT), jnp.int32),
            pltpu.VMEM((CHUNK_ROWS, D), jnp.float32),
            pltpu.VMEM((RPW, D), jnp.float32),
            pltpu.VMEM_SHARED((NS * RPW, D), jnp.float32),
            pltpu.SemaphoreType.DMA,
            pltpu.SemaphoreType.DMA,
        ],
        compiler_params=pltpu.CompilerParams(use_tc_tiling_on_sc=False),
    )(x1, table)


def _mm_body(p_ref, w_ref, b_ref, o_ref):
    p = p_ref[...] * (1.0 / SEQ)
    o_ref[...] = lax.dot_general(
        p, w_ref[...], (((1,), (1,)), ((), ())),
        preferred_element_type=jnp.float32) + b_ref[...]


def _classifier(pooled_sums, W, b):
    return pl.pallas_call(
        _mm_body,
        out_shape=jax.ShapeDtypeStruct((B, OUT_DIM), jnp.float32),
    )(pooled_sums, W, b.reshape(1, OUT_DIM))


def kernel(x, emb_table, W, b):
    x1 = x.astype(jnp.int32).reshape(B * SEQ)
    pooled_sums = _sc_pooled_sums(x1, emb_table)
    return _classifier(pooled_sums, W, b)
